# Initial kernel scaffold; baseline (speedup 1.0000x reference)
import jax
import jax.numpy as jnp
from jax.experimental import pallas as pl


def kernel(support, edge_index, edge_weight, W):
    raise NotImplementedError("write your pallas kernel here")


# speedup vs baseline: 4.5350x; 4.5350x over previous
import functools

import jax
import jax.numpy as jnp
from jax import lax
from jax.experimental import pallas as pl
from jax.experimental.pallas import tpu as pltpu
from jax.experimental.pallas import tpu_sc as plsc

NC = 2
NS = 16
NW = NC * NS


def _sc_aggregate(support, src, dst, ew):
    N, D = support.shape
    E = src.shape[0]
    assert E % NW == 0
    epw = E // NW
    CH = 80
    assert epw % CH == 0
    nchunk = epw // CH
    assert N % NS == 0
    rows_per_tile = N // NS
    ZR = 125
    assert rows_per_tile % ZR == 0
    nseg = D // 16

    mesh = plsc.VectorSubcoreMesh(core_axis_name="c", subcore_axis_name="s")

    @functools.partial(
        pl.kernel,
        mesh=mesh,
        out_type=jax.ShapeDtypeStruct((NC, N, D), jnp.float32),
        scratch_types=[
            pltpu.VMEM((CH,), jnp.int32),
            pltpu.VMEM((CH,), jnp.int32),
            pltpu.VMEM((CH,), jnp.float32),
            pltpu.VMEM((CH, D), jnp.float32),
            pltpu.VMEM((ZR, D), jnp.float32),
            pltpu.VMEM_SHARED((N, D), jnp.float32),
            pltpu.SemaphoreType.DMA,
        ],
    )
    def body(support_hbm, src_hbm, dst_hbm, ew_hbm, out_hbm,
             src_v, dst_v, w_v, rows_v, zero_v, acc_sh, sem):
        cid = lax.axis_index("c")
        sid = lax.axis_index("s")
        wid = sid * NC + cid

        zvec = jnp.zeros((16,), jnp.float32)

        def zero_row(i, carry):
            for p in range(nseg):
                zero_v[i, pl.ds(p * 16, 16)] = zvec
            return carry

        lax.fori_loop(0, ZR, zero_row, 0)
        for k in range(rows_per_tile // ZR):
            pltpu.sync_copy(
                zero_v, acc_sh.at[pl.ds(sid * rows_per_tile + k * ZR, ZR)])
        plsc.subcore_barrier()

        def chunk_body(i, carry):
            base = wid * epw + i * CH
            pltpu.sync_copy(src_hbm.at[pl.ds(base, CH)], src_v)
            pltpu.sync_copy(dst_hbm.at[pl.ds(base, CH)], dst_v)
            pltpu.sync_copy(ew_hbm.at[pl.ds(base, CH)], w_v)
            pltpu.async_copy(support_hbm.at[src_v], rows_v, sem).wait()

            def scale_group(g, c2):
                wvec = w_v[pl.ds(g * 16, 16)]
                for l in range(16):
                    w = wvec[l]
                    j = g * 16 + l
                    for p in range(nseg):
                        sl = pl.ds(p * 16, 16)
                        rows_v[j, sl] = rows_v[j, sl] * w
                return c2

            lax.fori_loop(0, CH // 16, scale_group, 0)
            pltpu.sync_copy(rows_v, acc_sh.at[dst_v], add=True)
            return carry

        lax.fori_loop(0, nchunk, chunk_body, 0)
        plsc.subcore_barrier()

        main = (N // 8 // NS) * 8
        r0 = sid * main
        pltpu.sync_copy(acc_sh.at[pl.ds(r0, main)],
                        out_hbm.at[cid, pl.ds(r0, main)])
        rem = N - main * NS
        if rem:
            nrem = rem // 8

            @pl.when(sid < nrem)
            def _():
                rr = main * NS + sid * 8
                pltpu.sync_copy(acc_sh.at[pl.ds(rr, 8)],
                                out_hbm.at[cid, pl.ds(rr, 8)])

    return body(support, src, dst, ew)


def _tc_combine(partials, W):
    _, N, D = partials.shape
    DO = W.shape[1]
    BLK = 1000
    assert N % BLK == 0

    def body(p_ref, w_ref, o_ref):
        s = p_ref[0] + p_ref[1]
        o_ref[...] = jnp.dot(s, w_ref[...], preferred_element_type=jnp.float32)

    return pl.pallas_call(
        body,
        grid=(N // BLK,),
        in_specs=[
            pl.BlockSpec((2, BLK, D), lambda i: (0, i, 0)),
            pl.BlockSpec((D, DO), lambda i: (0, 0)),
        ],
        out_specs=pl.BlockSpec((BLK, DO), lambda i: (i, 0)),
        out_shape=jax.ShapeDtypeStruct((N, DO), jnp.float32),
    )(partials, W)


def kernel(support, edge_index, edge_weight, W):
    dst = edge_index[0].astype(jnp.int32)
    src = edge_index[1].astype(jnp.int32)
    partials = _sc_aggregate(support, src, dst, edge_weight)
    return _tc_combine(partials, W)
